# Initial kernel scaffold; baseline (speedup 1.0000x reference)
import functools

import jax
import jax.numpy as jnp
from jax import lax
from jax.experimental import pallas as pl
from jax.experimental.pallas import tpu as pltpu
from jax.experimental.pallas import tpu_sc as plsc

N_NODES = 10000
N_EDGES = 320000
D = 128

NC = 2
NS = 16
NW = NC * NS

EDGES_PER_TILE = N_EDGES // NW
CHUNK = 80
NSTEPS = EDGES_PER_TILE // CHUNK

ROWS_PER_TILE = N_NODES // NS
ZROWS = 125


def _sc_segment_sum(x, src, dst):
    mesh = plsc.VectorSubcoreMesh(core_axis_name="c", subcore_axis_name="s")

    @functools.partial(
        pl.kernel,
        mesh=mesh,
        out_type=jax.ShapeDtypeStruct((NC * N_NODES, D), jnp.float32),
        scratch_types=[
            pltpu.VMEM((CHUNK,), jnp.int32),
            pltpu.VMEM((CHUNK,), jnp.int32),
            pltpu.VMEM((CHUNK, D), jnp.float32),
            pltpu.VMEM((ZROWS, D), jnp.float32),
            pltpu.VMEM_SHARED((N_NODES, D), jnp.float32),
            pltpu.SemaphoreType.DMA,
        ],
    )
    def seg_sum(x_hbm, src_hbm, dst_hbm, out_hbm, srcbuf, dstbuf, rows, zbuf,
                acc, sem):
        c = lax.axis_index("c")
        s = lax.axis_index("s")

        zero = jnp.zeros((16,), jnp.float32)

        def zstep(i, carry):
            r = i // (D // 16)
            col = (i % (D // 16)) * 16
            zbuf[r, pl.ds(col, 16)] = zero
            return carry

        lax.fori_loop(0, ZROWS * (D // 16), zstep, 0)

        for kk in range(ROWS_PER_TILE // ZROWS):
            pltpu.sync_copy(
                zbuf, acc.at[pl.ds(s * ROWS_PER_TILE + kk * ZROWS, ZROWS)])
        plsc.subcore_barrier()

        base0 = c * (N_EDGES // NC) + s * EDGES_PER_TILE

        def step(j, carry):
            base = base0 + j * CHUNK
            pltpu.sync_copy(src_hbm.at[pl.ds(base, CHUNK)], srcbuf)
            pltpu.sync_copy(dst_hbm.at[pl.ds(base, CHUNK)], dstbuf)
            pltpu.async_copy(x_hbm.at[srcbuf], rows, sem).wait()
            pltpu.sync_copy(rows, acc.at[dstbuf], add=True)
            return carry

        lax.fori_loop(0, NSTEPS, step, 0)
        plsc.subcore_barrier()

        for kk in range(ROWS_PER_TILE // ZROWS):
            r0 = s * ROWS_PER_TILE + kk * ZROWS
            pltpu.sync_copy(acc.at[pl.ds(r0, ZROWS)], zbuf)
            pltpu.sync_copy(zbuf, out_hbm.at[pl.ds(c * N_NODES + r0, ZROWS)])

    return seg_sum(x, src, dst)


def _mlp(eps, x, p0, p1, W1t, b1, gamma, beta, W2t, b2):
    BLK = 1000

    def body(eps_ref, x_ref, p0_ref, p1_ref, W1_ref, b1_ref, g_ref, be_ref,
             W2_ref, b2_ref, o_ref):
        h = x_ref[...] * (1.0 + eps_ref[0]) + p0_ref[...] + p1_ref[...]
        h = jnp.dot(h, W1_ref[...], preferred_element_type=jnp.float32)
        h = h + b1_ref[...]
        mu = jnp.mean(h, axis=-1, keepdims=True)
        hc = h - mu
        var = jnp.mean(hc * hc, axis=-1, keepdims=True)
        h = hc * lax.rsqrt(var + 1e-5) * g_ref[...] + be_ref[...]
        h = jnp.maximum(h, 0.0)
        o_ref[...] = (
            jnp.dot(h, W2_ref[...], preferred_element_type=jnp.float32)
            + b2_ref[...])

    full = lambda i: (0, 0)
    return pl.pallas_call(
        body,
        grid=(N_NODES // BLK,),
        in_specs=[
            pl.BlockSpec(memory_space=pltpu.SMEM),
            pl.BlockSpec((BLK, D), lambda i: (i, 0)),
            pl.BlockSpec((BLK, D), lambda i: (i, 0)),
            pl.BlockSpec((BLK, D), lambda i: (i, 0)),
            pl.BlockSpec((D, D), full),
            pl.BlockSpec((1, D), full),
            pl.BlockSpec((1, D), full),
            pl.BlockSpec((1, D), full),
            pl.BlockSpec((D, D), full),
            pl.BlockSpec((1, D), full),
        ],
        out_specs=pl.BlockSpec((BLK, D), lambda i: (i, 0)),
        out_shape=jax.ShapeDtypeStruct((N_NODES, D), jnp.float32),
    )(eps, x, p0, p1, W1t, b1, gamma, beta, W2t, b2)


def kernel(x, edge_index, eps, W1, b1, gamma, beta, W2, b2):
    src = edge_index[0].astype(jnp.int32)
    dst = edge_index[1].astype(jnp.int32)
    parts = _sc_segment_sum(x, src, dst)
    p0 = parts[:N_NODES]
    p1 = parts[N_NODES:]
    return _mlp(
        eps.reshape(1), x, p0, p1,
        W1.T, b1.reshape(1, D), gamma.reshape(1, D), beta.reshape(1, D),
        W2.T, b2.reshape(1, D))


# speedup vs baseline: 5.4339x; 5.4339x over previous
import functools

import jax
import jax.numpy as jnp
from jax import lax
from jax.experimental import pallas as pl
from jax.experimental.pallas import tpu as pltpu
from jax.experimental.pallas import tpu_sc as plsc

N_NODES = 10000
N_EDGES = 320000
D = 128

NC = 2
NS = 16
NW = NC * NS

EDGES_PER_TILE = N_EDGES // NW
CHUNK = 80
NSTEPS = EDGES_PER_TILE // CHUNK

RCHUNK = 80
NRCHUNKS = N_NODES // RCHUNK


def _sc_segment_sum(x, src, dst):
    mesh = plsc.VectorSubcoreMesh(core_axis_name="c", subcore_axis_name="s")

    @functools.partial(
        pl.kernel,
        mesh=mesh,
        out_type=jax.ShapeDtypeStruct((NC * N_NODES, D), jnp.float32),
        scratch_types=[
            pltpu.VMEM((CHUNK,), jnp.int32),
            pltpu.VMEM((CHUNK,), jnp.int32),
            pltpu.VMEM((CHUNK, D), jnp.float32),
            pltpu.VMEM_SHARED((N_NODES, D), jnp.float32),
            pltpu.SemaphoreType.DMA,
        ],
    )
    def seg_sum(x_hbm, src_hbm, dst_hbm, out_hbm, srcbuf, dstbuf, rows,
                acc, sem):
        c = lax.axis_index("c")
        s = lax.axis_index("s")

        zero = jnp.zeros((16,), jnp.float32)

        def zstep(i, carry):
            r = i // (D // 16)
            col = (i % (D // 16)) * 16
            rows[r, pl.ds(col, 16)] = zero
            return carry

        lax.fori_loop(0, CHUNK * (D // 16), zstep, 0)

        def zcopy(kk, carry):
            chunk = kk * NS + s
            @pl.when(chunk < NRCHUNKS)
            def _():
                pltpu.sync_copy(rows, acc.at[pl.ds(chunk * RCHUNK, RCHUNK)])
            return carry

        lax.fori_loop(0, (NRCHUNKS + NS - 1) // NS, zcopy, 0)
        plsc.subcore_barrier()

        base0 = c * (N_EDGES // NC) + s * EDGES_PER_TILE

        def step(j, carry):
            base = base0 + j * CHUNK
            pltpu.sync_copy(src_hbm.at[pl.ds(base, CHUNK)], srcbuf)
            pltpu.sync_copy(dst_hbm.at[pl.ds(base, CHUNK)], dstbuf)
            pltpu.async_copy(x_hbm.at[srcbuf], rows, sem).wait()
            pltpu.sync_copy(rows, acc.at[dstbuf], add=True)
            return carry

        lax.fori_loop(0, NSTEPS, step, 0)
        plsc.subcore_barrier()

        def dcopy(kk, carry):
            chunk = kk * NS + s
            @pl.when(chunk < NRCHUNKS)
            def _():
                r0 = chunk * RCHUNK
                pltpu.sync_copy(acc.at[pl.ds(r0, RCHUNK)], rows)
                pltpu.sync_copy(
                    rows, out_hbm.at[pl.ds(c * N_NODES + r0, RCHUNK)])
            return carry

        lax.fori_loop(0, (NRCHUNKS + NS - 1) // NS, dcopy, 0)

    return seg_sum(x, src, dst)


def _mlp(eps, x, p0, p1, W1t, b1, gamma, beta, W2t, b2):
    BLK = 1000

    def body(eps_ref, x_ref, p0_ref, p1_ref, W1_ref, b1_ref, g_ref, be_ref,
             W2_ref, b2_ref, o_ref):
        h = x_ref[...] * (1.0 + eps_ref[0]) + p0_ref[...] + p1_ref[...]
        h = jnp.dot(h, W1_ref[...], preferred_element_type=jnp.float32)
        h = h + b1_ref[...]
        mu = jnp.mean(h, axis=-1, keepdims=True)
        hc = h - mu
        var = jnp.mean(hc * hc, axis=-1, keepdims=True)
        h = hc * lax.rsqrt(var + 1e-5) * g_ref[...] + be_ref[...]
        h = jnp.maximum(h, 0.0)
        o_ref[...] = (
            jnp.dot(h, W2_ref[...], preferred_element_type=jnp.float32)
            + b2_ref[...])

    full = lambda i: (0, 0)
    return pl.pallas_call(
        body,
        grid=(N_NODES // BLK,),
        in_specs=[
            pl.BlockSpec(memory_space=pltpu.SMEM),
            pl.BlockSpec((BLK, D), lambda i: (i, 0)),
            pl.BlockSpec((BLK, D), lambda i: (i, 0)),
            pl.BlockSpec((BLK, D), lambda i: (i, 0)),
            pl.BlockSpec((D, D), full),
            pl.BlockSpec((1, D), full),
            pl.BlockSpec((1, D), full),
            pl.BlockSpec((1, D), full),
            pl.BlockSpec((D, D), full),
            pl.BlockSpec((1, D), full),
        ],
        out_specs=pl.BlockSpec((BLK, D), lambda i: (i, 0)),
        out_shape=jax.ShapeDtypeStruct((N_NODES, D), jnp.float32),
    )(eps, x, p0, p1, W1t, b1, gamma, beta, W2t, b2)


def kernel(x, edge_index, eps, W1, b1, gamma, beta, W2, b2):
    src = edge_index[0].astype(jnp.int32)
    dst = edge_index[1].astype(jnp.int32)
    parts = _sc_segment_sum(x, src, dst)
    p0 = parts[:N_NODES]
    p1 = parts[N_NODES:]
    return _mlp(
        eps.reshape(1), x, p0, p1,
        W1.T, b1.reshape(1, D), gamma.reshape(1, D), beta.reshape(1, D),
        W2.T, b2.reshape(1, D))


# speedup vs baseline: 9.5717x; 1.7615x over previous
import functools

import jax
import jax.numpy as jnp
from jax import lax
from jax.experimental import pallas as pl
from jax.experimental.pallas import tpu as pltpu
from jax.experimental.pallas import tpu_sc as plsc

N_NODES = 10000
N_EDGES = 320000
D = 128

NC = 2
NS = 16
NW = NC * NS

EDGES_PER_TILE = N_EDGES // NW
CHUNK = 80
NSTEPS = EDGES_PER_TILE // CHUNK

RCHUNK = 80
NRCHUNKS = N_NODES // RCHUNK


def _sc_segment_sum(x, src, dst):
    mesh = plsc.VectorSubcoreMesh(core_axis_name="c", subcore_axis_name="s")

    @functools.partial(
        pl.kernel,
        mesh=mesh,
        out_type=jax.ShapeDtypeStruct((NC * N_NODES, D), jnp.float32),
        scratch_types=[
            pltpu.VMEM((EDGES_PER_TILE,), jnp.int32),
            pltpu.VMEM((CHUNK,), jnp.int32),
            pltpu.VMEM((CHUNK,), jnp.int32),
            pltpu.VMEM((CHUNK, D), jnp.float32),
            pltpu.VMEM((CHUNK, D), jnp.float32),
            pltpu.VMEM_SHARED((N_NODES, D), jnp.float32),
            pltpu.SemaphoreType.DMA,
            pltpu.SemaphoreType.DMA,
            pltpu.SemaphoreType.DMA,
            pltpu.SemaphoreType.DMA,
            pltpu.SemaphoreType.DMA,
            pltpu.SemaphoreType.DMA,
        ],
    )
    def seg_sum(x_hbm, src_hbm, dst_hbm, out_hbm, src_all, dstA, dstB,
                rowsA, rowsB, acc, gA, gB, sA, sB, iA, iB):
        c = lax.axis_index("c")
        s = lax.axis_index("s")
        base0 = c * (N_EDGES // NC) + s * EDGES_PER_TILE

        pltpu.sync_copy(src_hbm.at[pl.ds(base0, EDGES_PER_TILE)], src_all)

        zero = jnp.zeros((16,), jnp.float32)

        def zstep(i, carry):
            r = i // (D // 16)
            col = (i % (D // 16)) * 16
            rowsA[r, pl.ds(col, 16)] = zero
            return carry

        lax.fori_loop(0, CHUNK * (D // 16), zstep, 0)

        def zcopy(kk, carry):
            chunk = kk * NS + s
            @pl.when(chunk < NRCHUNKS)
            def _():
                pltpu.sync_copy(rowsA, acc.at[pl.ds(chunk * RCHUNK, RCHUNK)])
            return carry

        lax.fori_loop(0, (NRCHUNKS + NS - 1) // NS, zcopy, 0)
        plsc.subcore_barrier()

        dstbuf = (dstA, dstB)
        rows = (rowsA, rowsB)
        gsem = (gA, gB)
        ssem = (sA, sB)
        isem = (iA, iB)

        def issue(j, b):
            pltpu.async_copy(
                dst_hbm.at[pl.ds(base0 + j * CHUNK, CHUNK)], dstbuf[b],
                isem[b])
            pltpu.async_copy(
                x_hbm.at[src_all.at[pl.ds(j * CHUNK, CHUNK)]], rows[b],
                gsem[b])

        def wait_in(b):
            pltpu.make_async_copy(
                dst_hbm.at[pl.ds(base0, CHUNK)], dstbuf[b], isem[b]).wait()
            pltpu.make_async_copy(
                x_hbm.at[src_all.at[pl.ds(0, CHUNK)]], rows[b],
                gsem[b]).wait()

        def scatter(b):
            pltpu.async_copy(rows[b], acc.at[dstbuf[b]], ssem[b], add=True)

        def wait_scatter(b):
            pltpu.make_async_copy(rows[b], acc.at[dstbuf[b]], ssem[b]).wait()

        issue(0, 0)
        issue(1, 1)

        def body(i, carry):
            j0 = 2 * i
            j1 = 2 * i + 1
            wait_in(0)
            scatter(0)

            @pl.when(j1 < NSTEPS)
            def _():
                wait_in(1)
                scatter(1)

            @pl.when(j0 + 2 < NSTEPS)
            def _():
                wait_scatter(0)
                issue(j0 + 2, 0)

            @pl.when(j1 + 2 < NSTEPS)
            def _():
                wait_scatter(1)
                issue(j1 + 2, 1)

            return carry

        lax.fori_loop(0, (NSTEPS + 1) // 2, body, 0)
        wait_scatter(0)
        wait_scatter(1)
        plsc.subcore_barrier()

        def dcopy(kk, carry):
            chunk = kk * NS + s
            @pl.when(chunk < NRCHUNKS)
            def _():
                r0 = chunk * RCHUNK
                pltpu.sync_copy(acc.at[pl.ds(r0, RCHUNK)], rowsA)
                pltpu.sync_copy(
                    rowsA, out_hbm.at[pl.ds(c * N_NODES + r0, RCHUNK)])
            return carry

        lax.fori_loop(0, (NRCHUNKS + NS - 1) // NS, dcopy, 0)

    return seg_sum(x, src, dst)


def _mlp(eps, x, p0, p1, W1t, b1, gamma, beta, W2t, b2):
    BLK = 1000

    def body(eps_ref, x_ref, p0_ref, p1_ref, W1_ref, b1_ref, g_ref, be_ref,
             W2_ref, b2_ref, o_ref):
        h = x_ref[...] * (1.0 + eps_ref[0]) + p0_ref[...] + p1_ref[...]
        h = jnp.dot(h, W1_ref[...], preferred_element_type=jnp.float32)
        h = h + b1_ref[...]
        mu = jnp.mean(h, axis=-1, keepdims=True)
        hc = h - mu
        var = jnp.mean(hc * hc, axis=-1, keepdims=True)
        h = hc * lax.rsqrt(var + 1e-5) * g_ref[...] + be_ref[...]
        h = jnp.maximum(h, 0.0)
        o_ref[...] = (
            jnp.dot(h, W2_ref[...], preferred_element_type=jnp.float32)
            + b2_ref[...])

    full = lambda i: (0, 0)
    return pl.pallas_call(
        body,
        grid=(N_NODES // BLK,),
        in_specs=[
            pl.BlockSpec(memory_space=pltpu.SMEM),
            pl.BlockSpec((BLK, D), lambda i: (i, 0)),
            pl.BlockSpec((BLK, D), lambda i: (i, 0)),
            pl.BlockSpec((BLK, D), lambda i: (i, 0)),
            pl.BlockSpec((D, D), full),
            pl.BlockSpec((1, D), full),
            pl.BlockSpec((1, D), full),
            pl.BlockSpec((1, D), full),
            pl.BlockSpec((D, D), full),
            pl.BlockSpec((1, D), full),
        ],
        out_specs=pl.BlockSpec((BLK, D), lambda i: (i, 0)),
        out_shape=jax.ShapeDtypeStruct((N_NODES, D), jnp.float32),
    )(eps, x, p0, p1, W1t, b1, gamma, beta, W2t, b2)


def kernel(x, edge_index, eps, W1, b1, gamma, beta, W2, b2):
    src = edge_index[0].astype(jnp.int32)
    dst = edge_index[1].astype(jnp.int32)
    parts = _sc_segment_sum(x, src, dst)
    p0 = parts[:N_NODES]
    p1 = parts[N_NODES:]
    return _mlp(
        eps.reshape(1), x, p0, p1,
        W1.T, b1.reshape(1, D), gamma.reshape(1, D), beta.reshape(1, D),
        W2.T, b2.reshape(1, D))


# speedup vs baseline: 12.0410x; 1.2580x over previous
import functools

import jax
import jax.numpy as jnp
from jax import lax
from jax.experimental import pallas as pl
from jax.experimental.pallas import tpu as pltpu
from jax.experimental.pallas import tpu_sc as plsc

N_NODES = 10000
N_EDGES = 320000
D = 128

NC = 2
NS = 16
NW = NC * NS

EDGES_PER_TILE = N_EDGES // NW
CHUNK = 80
NSTEPS = EDGES_PER_TILE // CHUNK
NBUF = 3

RCHUNK = 80
NRCHUNKS = N_NODES // RCHUNK


def _sc_segment_sum(x, src, dst):
    mesh = plsc.VectorSubcoreMesh(core_axis_name="c", subcore_axis_name="s")

    @functools.partial(
        pl.kernel,
        mesh=mesh,
        out_type=jax.ShapeDtypeStruct((NC * N_NODES, D), jnp.float32),
        scratch_types=(
            [pltpu.VMEM((EDGES_PER_TILE,), jnp.int32)]
            + [pltpu.VMEM((CHUNK,), jnp.int32) for _ in range(NBUF)]
            + [pltpu.VMEM((CHUNK, D), jnp.float32) for _ in range(NBUF)]
            + [pltpu.VMEM_SHARED((N_NODES, D), jnp.float32)]
            + [pltpu.SemaphoreType.DMA for _ in range(3 * NBUF)]
        ),
    )
    def seg_sum(x_hbm, src_hbm, dst_hbm, out_hbm, src_all, *rest):
        dstbuf = rest[:NBUF]
        rows = rest[NBUF:2 * NBUF]
        acc = rest[2 * NBUF]
        gsem = rest[2 * NBUF + 1:2 * NBUF + 1 + NBUF]
        ssem = rest[2 * NBUF + 1 + NBUF:2 * NBUF + 1 + 2 * NBUF]
        isem = rest[2 * NBUF + 1 + 2 * NBUF:]
        rowsA = rows[0]
        c = lax.axis_index("c")
        s = lax.axis_index("s")
        base0 = c * (N_EDGES // NC) + s * EDGES_PER_TILE

        pltpu.sync_copy(src_hbm.at[pl.ds(base0, EDGES_PER_TILE)], src_all)

        zero = jnp.zeros((16,), jnp.float32)

        def zstep(i, carry):
            r = i // (D // 16)
            col = (i % (D // 16)) * 16
            rowsA[r, pl.ds(col, 16)] = zero
            return carry

        lax.fori_loop(0, CHUNK * (D // 16), zstep, 0)

        def zcopy(kk, carry):
            chunk = kk * NS + s
            @pl.when(chunk < NRCHUNKS)
            def _():
                pltpu.sync_copy(rowsA, acc.at[pl.ds(chunk * RCHUNK, RCHUNK)])
            return carry

        lax.fori_loop(0, (NRCHUNKS + NS - 1) // NS, zcopy, 0)
        plsc.subcore_barrier()

        def issue(j, b):
            pltpu.async_copy(
                dst_hbm.at[pl.ds(base0 + j * CHUNK, CHUNK)], dstbuf[b],
                isem[b])
            pltpu.async_copy(
                x_hbm.at[src_all.at[pl.ds(j * CHUNK, CHUNK)]], rows[b],
                gsem[b])

        def wait_in(b):
            pltpu.make_async_copy(
                dst_hbm.at[pl.ds(base0, CHUNK)], dstbuf[b], isem[b]).wait()
            pltpu.make_async_copy(
                x_hbm.at[src_all.at[pl.ds(0, CHUNK)]], rows[b],
                gsem[b]).wait()

        def scatter(b):
            pltpu.async_copy(rows[b], acc.at[dstbuf[b]], ssem[b], add=True)

        def wait_scatter(b):
            pltpu.make_async_copy(rows[b], acc.at[dstbuf[b]], ssem[b]).wait()

        for b in range(NBUF):
            issue(b, b)

        def body(i, carry):
            for b in range(NBUF):
                j = NBUF * i + b

                @pl.when(j < NSTEPS)
                def _(b=b):
                    wait_in(b)
                    scatter(b)

            for b in range(NBUF):
                j = NBUF * i + b

                @pl.when(j + NBUF < NSTEPS)
                def _(b=b, j=j):
                    wait_scatter(b)
                    issue(j + NBUF, b)

            return carry

        lax.fori_loop(0, (NSTEPS + NBUF - 1) // NBUF, body, 0)
        for b in range(NBUF):
            wait_scatter(b)
        plsc.subcore_barrier()

        def dcopy(kk, carry):
            chunk = kk * NS + s
            @pl.when(chunk < NRCHUNKS)
            def _():
                r0 = chunk * RCHUNK
                pltpu.sync_copy(acc.at[pl.ds(r0, RCHUNK)], rowsA)
                pltpu.sync_copy(
                    rowsA, out_hbm.at[pl.ds(c * N_NODES + r0, RCHUNK)])
            return carry

        lax.fori_loop(0, (NRCHUNKS + NS - 1) // NS, dcopy, 0)

    return seg_sum(x, src, dst)


def _mlp(eps, x, parts, W1t, b1, gamma, beta, W2t, b2):
    BLK = 2000

    def body(eps_ref, x_ref, p0_ref, p1_ref, W1_ref, b1_ref, g_ref, be_ref,
             W2_ref, b2_ref, o_ref):
        h = x_ref[...] * (1.0 + eps_ref[0]) + p0_ref[...] + p1_ref[...]
        h = jnp.dot(h, W1_ref[...], preferred_element_type=jnp.float32)
        h = h + b1_ref[...]
        mu = jnp.mean(h, axis=-1, keepdims=True)
        hc = h - mu
        var = jnp.mean(hc * hc, axis=-1, keepdims=True)
        h = hc * lax.rsqrt(var + 1e-5) * g_ref[...] + be_ref[...]
        h = jnp.maximum(h, 0.0)
        o_ref[...] = (
            jnp.dot(h, W2_ref[...], preferred_element_type=jnp.float32)
            + b2_ref[...])

    full = lambda i: (0, 0)
    nblk = N_NODES // BLK
    return pl.pallas_call(
        body,
        grid=(nblk,),
        in_specs=[
            pl.BlockSpec(memory_space=pltpu.SMEM),
            pl.BlockSpec((BLK, D), lambda i: (i, 0)),
            pl.BlockSpec((BLK, D), lambda i: (i, 0)),
            pl.BlockSpec((BLK, D), lambda i: (i + N_NODES // BLK, 0)),
            pl.BlockSpec((D, D), full),
            pl.BlockSpec((1, D), full),
            pl.BlockSpec((1, D), full),
            pl.BlockSpec((1, D), full),
            pl.BlockSpec((D, D), full),
            pl.BlockSpec((1, D), full),
        ],
        out_specs=pl.BlockSpec((BLK, D), lambda i: (i, 0)),
        out_shape=jax.ShapeDtypeStruct((N_NODES, D), jnp.float32),
    )(eps, x, parts, parts, W1t, b1, gamma, beta, W2t, b2)


def kernel(x, edge_index, eps, W1, b1, gamma, beta, W2, b2):
    src = edge_index[0].astype(jnp.int32)
    dst = edge_index[1].astype(jnp.int32)
    parts = _sc_segment_sum(x, src, dst)
    return _mlp(
        eps.reshape(1), x, parts,
        W1.T, b1.reshape(1, D), gamma.reshape(1, D), beta.reshape(1, D),
        W2.T, b2.reshape(1, D))


# speedup vs baseline: 12.2540x; 1.0177x over previous
import functools

import jax
import jax.numpy as jnp
from jax import lax
from jax.experimental import pallas as pl
from jax.experimental.pallas import tpu as pltpu
from jax.experimental.pallas import tpu_sc as plsc

N_NODES = 10000
N_EDGES = 320000
D = 128

NC = 2
NS = 16
NW = NC * NS

EDGES_PER_TILE = N_EDGES // NW
CHUNK = 80
NSTEPS = EDGES_PER_TILE // CHUNK
NBUF = 3

RCHUNK = 80
NRCHUNKS = N_NODES // RCHUNK


def _sc_segment_sum(x, src, dst):
    mesh = plsc.VectorSubcoreMesh(core_axis_name="c", subcore_axis_name="s")

    @functools.partial(
        pl.kernel,
        mesh=mesh,
        out_type=jax.ShapeDtypeStruct((NC * N_NODES, D), jnp.float32),
        scratch_types=(
            [pltpu.VMEM((EDGES_PER_TILE,), jnp.int32)]
            + [pltpu.VMEM((CHUNK,), jnp.int32) for _ in range(NBUF)]
            + [pltpu.VMEM((CHUNK, D), jnp.float32) for _ in range(NBUF)]
            + [pltpu.VMEM_SHARED((N_NODES, D), jnp.float32)]
            + [pltpu.SemaphoreType.DMA for _ in range(3 * NBUF)]
        ),
    )
    def seg_sum(x_hbm, src_hbm, dst_hbm, out_hbm, src_all, *rest):
        dstbuf = rest[:NBUF]
        rows = rest[NBUF:2 * NBUF]
        acc = rest[2 * NBUF]
        gsem = rest[2 * NBUF + 1:2 * NBUF + 1 + NBUF]
        ssem = rest[2 * NBUF + 1 + NBUF:2 * NBUF + 1 + 2 * NBUF]
        isem = rest[2 * NBUF + 1 + 2 * NBUF:]
        rowsA = rows[0]
        c = lax.axis_index("c")
        s = lax.axis_index("s")
        base0 = c * (N_EDGES // NC) + s * EDGES_PER_TILE

        pltpu.sync_copy(src_hbm.at[pl.ds(base0, EDGES_PER_TILE)], src_all)

        zero = jnp.zeros((16,), jnp.float32)

        def zstep(i, carry):
            r = i // (D // 16)
            col = (i % (D // 16)) * 16
            rowsA[r, pl.ds(col, 16)] = zero
            return carry

        lax.fori_loop(0, CHUNK * (D // 16), zstep, 0)

        def zcopy(kk, carry):
            chunk = kk * NS + s
            @pl.when(chunk < NRCHUNKS)
            def _():
                pltpu.sync_copy(rowsA, acc.at[pl.ds(chunk * RCHUNK, RCHUNK)])
            return carry

        lax.fori_loop(0, (NRCHUNKS + NS - 1) // NS, zcopy, 0)
        plsc.subcore_barrier()

        def issue(j, b):
            pltpu.async_copy(
                dst_hbm.at[pl.ds(base0 + j * CHUNK, CHUNK)], dstbuf[b],
                isem[b])
            pltpu.async_copy(
                x_hbm.at[src_all.at[pl.ds(j * CHUNK, CHUNK)]], rows[b],
                gsem[b])

        def wait_in(b):
            pltpu.make_async_copy(
                dst_hbm.at[pl.ds(base0, CHUNK)], dstbuf[b], isem[b]).wait()
            pltpu.make_async_copy(
                x_hbm.at[src_all.at[pl.ds(0, CHUNK)]], rows[b],
                gsem[b]).wait()

        def scatter(b):
            pltpu.async_copy(rows[b], acc.at[dstbuf[b]], ssem[b], add=True)

        def wait_scatter(b):
            pltpu.make_async_copy(rows[b], acc.at[dstbuf[b]], ssem[b]).wait()

        for b in range(NBUF):
            issue(b, b)

        def body(i, carry):
            for b in range(NBUF):
                j = NBUF * i + b

                @pl.when(j < NSTEPS)
                def _(b=b):
                    wait_in(b)
                    scatter(b)

            for b in range(NBUF):
                j = NBUF * i + b

                @pl.when(j + NBUF < NSTEPS)
                def _(b=b, j=j):
                    wait_scatter(b)
                    issue(j + NBUF, b)

            return carry

        lax.fori_loop(0, (NSTEPS + NBUF - 1) // NBUF, body, 0)
        for b in range(NBUF):
            wait_scatter(b)
        plsc.subcore_barrier()

        def dcopy(kk, carry):
            chunk = kk * NS + s
            @pl.when(chunk < NRCHUNKS)
            def _():
                r0 = chunk * RCHUNK
                pltpu.sync_copy(
                    acc.at[pl.ds(r0, RCHUNK)],
                    out_hbm.at[pl.ds(c * N_NODES + r0, RCHUNK)])
            return carry

        lax.fori_loop(0, (NRCHUNKS + NS - 1) // NS, dcopy, 0)

    return seg_sum(x, src, dst)


def _mlp(eps, x, parts, W1t, b1, gamma, beta, W2t, b2):
    BLK = 2000

    def body(eps_ref, x_ref, p0_ref, p1_ref, W1_ref, b1_ref, g_ref, be_ref,
             W2_ref, b2_ref, o_ref):
        h = x_ref[...] * (1.0 + eps_ref[0]) + p0_ref[...] + p1_ref[...]
        h = jnp.dot(h, W1_ref[...], preferred_element_type=jnp.float32)
        h = h + b1_ref[...]
        mu = jnp.mean(h, axis=-1, keepdims=True)
        hc = h - mu
        var = jnp.mean(hc * hc, axis=-1, keepdims=True)
        h = hc * lax.rsqrt(var + 1e-5) * g_ref[...] + be_ref[...]
        h = jnp.maximum(h, 0.0)
        o_ref[...] = (
            jnp.dot(h, W2_ref[...], preferred_element_type=jnp.float32)
            + b2_ref[...])

    full = lambda i: (0, 0)
    nblk = N_NODES // BLK
    return pl.pallas_call(
        body,
        grid=(nblk,),
        in_specs=[
            pl.BlockSpec(memory_space=pltpu.SMEM),
            pl.BlockSpec((BLK, D), lambda i: (i, 0)),
            pl.BlockSpec((BLK, D), lambda i: (i, 0)),
            pl.BlockSpec((BLK, D), lambda i: (i + N_NODES // BLK, 0)),
            pl.BlockSpec((D, D), full),
            pl.BlockSpec((1, D), full),
            pl.BlockSpec((1, D), full),
            pl.BlockSpec((1, D), full),
            pl.BlockSpec((D, D), full),
            pl.BlockSpec((1, D), full),
        ],
        out_specs=pl.BlockSpec((BLK, D), lambda i: (i, 0)),
        out_shape=jax.ShapeDtypeStruct((N_NODES, D), jnp.float32),
    )(eps, x, parts, parts, W1t, b1, gamma, beta, W2t, b2)


def kernel(x, edge_index, eps, W1, b1, gamma, beta, W2, b2):
    src = edge_index[0].astype(jnp.int32)
    dst = edge_index[1].astype(jnp.int32)
    parts = _sc_segment_sum(x, src, dst)
    return _mlp(
        eps.reshape(1), x, parts,
        W1.T, b1.reshape(1, D), gamma.reshape(1, D), beta.reshape(1, D),
        W2.T, b2.reshape(1, D))
